# Initial kernel scaffold; baseline (speedup 1.0000x reference)
import jax
import jax.numpy as jnp
from jax.experimental import pallas as pl


def kernel(x, params):
    raise NotImplementedError("write your pallas kernel here")


# speedup vs baseline: 7.0332x; 7.0332x over previous
import jax
import jax.numpy as jnp
from jax import lax
from jax.experimental import pallas as pl
from jax.experimental.pallas import tpu as pltpu
from jax.experimental.pallas import tpu_sc as plsc

N = 4096
KNB = 30
NE = N * KNB
EPS = 1e-5
F32 = jnp.float32

SC_NC = 2
SC_NS = 16
SC_NW = SC_NC * SC_NS
SC_CH = 128
SC_CPW = NE // (SC_NW * SC_CH)
SC_HALF = SC_CPW // 2


def _bn(v, mu, var, g, be):
    return (v - mu) / jnp.sqrt(var + EPS) * g + be


_KNN_RB = 256


def _knn_body(ft_ref, fb_ref, idx_ref, d_scr):
    i = pl.program_id(0)
    fb = fb_ref[...]
    ft = ft_ref[...]
    g = jnp.dot(fb, ft, preferred_element_type=F32)
    sqa = jnp.sum(ft * ft, axis=0, keepdims=True)
    sqb = jnp.sum(fb * fb, axis=1, keepdims=True)
    dist = sqb + sqa - 2.0 * g
    rows = i * _KNN_RB + lax.broadcasted_iota(jnp.int32, (_KNN_RB, N), 0)
    cols = lax.broadcasted_iota(jnp.int32, (_KNN_RB, N), 1)
    dist = dist + jnp.where(rows == cols, F32(1e10), F32(0.0))
    d_scr[...] = dist
    slot = lax.broadcasted_iota(jnp.int32, (_KNN_RB, 32), 1)

    def sel_body(t, acc):
        dc = d_scr[...]
        m = jnp.min(dc, axis=1, keepdims=True)
        cand = jnp.where(dc == m, cols, jnp.int32(N))
        sel = jnp.min(cand, axis=1, keepdims=True)
        d_scr[...] = jnp.where(cols == sel, F32(3e38), dc)
        return acc + sel * (slot == t).astype(jnp.int32)

    acc = lax.fori_loop(
        0, KNB, sel_body, jnp.zeros((_KNN_RB, 32), jnp.int32))
    idx_ref[...] = acc[:, :KNB]


def _knn(feat):
    d = feat.shape[1]
    return pl.pallas_call(
        _knn_body,
        grid=(N // _KNN_RB,),
        in_specs=[
            pl.BlockSpec((d, N), lambda i: (0, 0)),
            pl.BlockSpec((_KNN_RB, d), lambda i: (i, 0)),
        ],
        out_specs=pl.BlockSpec((_KNN_RB, KNB), lambda i: (i, 0)),
        out_shape=jax.ShapeDtypeStruct((N, KNB), jnp.int32),
        scratch_shapes=[pltpu.VMEM((_KNN_RB, N), F32)],
    )(feat.T, feat)


def _sc_gather_body(table_hbm, idx_hbm, out_hbm, idx_v, rows_v, sem):
    w = lax.axis_index("s") * SC_NC + lax.axis_index("c")
    pltpu.sync_copy(idx_hbm.at[w], idx_v)
    for half in range(2):
        cps = []
        for j in range(SC_HALF):
            t = half * SC_HALF + j
            cps.append(pltpu.async_copy(
                table_hbm.at[idx_v.at[t]],
                rows_v.at[pl.ds(j * SC_CH, SC_CH)], sem))
        for cp in cps:
            cp.wait()
        pltpu.sync_copy(
            rows_v,
            out_hbm.at[pl.ds(w * SC_CPW * SC_CH + half * SC_HALF * SC_CH,
                             SC_HALF * SC_CH)])


def _sc_gather(table, idx):
    c = table.shape[1]
    mesh = plsc.VectorSubcoreMesh(core_axis_name="c", subcore_axis_name="s")
    run = pl.kernel(
        _sc_gather_body,
        out_type=jax.ShapeDtypeStruct((NE, c), F32),
        mesh=mesh,
        compiler_params=pltpu.CompilerParams(use_tc_tiling_on_sc=False),
        scratch_types=[
            pltpu.VMEM((SC_CPW, SC_CH), jnp.int32),
            pltpu.VMEM((SC_HALF * SC_CH, c), F32),
            pltpu.SemaphoreType.DMA,
        ],
    )
    return run(table, idx.reshape(SC_NW, SC_CPW, SC_CH))


_EDGE_P = 128


def _edge_e(f_blk, vg_blk):
    gw = f_blk.shape[1]
    xi = jnp.broadcast_to(f_blk[:, None, :], (_EDGE_P, KNB, gw))
    xi = xi.reshape(_EDGE_P * KNB, gw)
    return jnp.concatenate([xi, vg_blk - xi], axis=1)


def _passA_body(f_ref, vg_ref, w1_ref, b1_ref, h_ref):
    e = _edge_e(f_ref[...], vg_ref[...])
    h_ref[...] = jnp.maximum(
        jnp.dot(e, w1_ref[...], preferred_element_type=F32) + b1_ref[...],
        0.0)


def _passA(featp, vg, W1p, b1):
    gw = featp.shape[1]
    c = W1p.shape[1]
    one = lambda i: (0, 0)
    return pl.pallas_call(
        _passA_body,
        grid=(N // _EDGE_P,),
        in_specs=[
            pl.BlockSpec((_EDGE_P, gw), lambda i: (i, 0)),
            pl.BlockSpec((_EDGE_P * KNB, gw), lambda i: (i, 0)),
            pl.BlockSpec((2 * gw, c), one),
            pl.BlockSpec((1, c), one),
        ],
        out_specs=pl.BlockSpec((_EDGE_P * KNB, c), lambda i: (i, 0)),
        out_shape=jax.ShapeDtypeStruct((NE, c), F32),
    )(featp, vg, W1p, b1.reshape(1, c))


def _passB_body(h1_ref, mu1_ref, var1_ref, g1_ref, be1_ref, w2_ref, b2_ref,
                h2_ref, mx_ref):
    z = _bn(h1_ref[...], mu1_ref[...], var1_ref[...], g1_ref[...],
            be1_ref[...])
    h2 = jnp.maximum(jnp.dot(z, w2_ref[...],
                             preferred_element_type=F32) + b2_ref[...], 0.0)
    c2 = h2.shape[1]
    h2_ref[...] = h2
    mx_ref[...] = jnp.max(h2.reshape(_EDGE_P, KNB, c2), axis=1)


def _passB(h1, mu1, var1, g1, be1, W2, b2):
    c = h1.shape[1]
    c2 = W2.shape[1]
    one = lambda i: (0, 0)
    return pl.pallas_call(
        _passB_body,
        grid=(N // _EDGE_P,),
        in_specs=[
            pl.BlockSpec((_EDGE_P * KNB, c), lambda i: (i, 0)),
            pl.BlockSpec((1, c), one), pl.BlockSpec((1, c), one),
            pl.BlockSpec((1, c), one), pl.BlockSpec((1, c), one),
            pl.BlockSpec((c, c2), one), pl.BlockSpec((1, c2), one),
        ],
        out_specs=(pl.BlockSpec((_EDGE_P * KNB, c2), lambda i: (i, 0)),
                   pl.BlockSpec((_EDGE_P, c2), lambda i: (i, 0))),
        out_shape=(jax.ShapeDtypeStruct((NE, c2), F32),
                   jax.ShapeDtypeStruct((N, c2), F32)),
    )(h1, mu1, var1, g1.reshape(1, c), be1.reshape(1, c), W2,
      b2.reshape(1, c2))


def _finish_body(mh_ref, mu_ref, var_ref, g_ref, be_ref, out_ref):
    out_ref[...] = _bn(mh_ref[...], mu_ref[...], var_ref[...], g_ref[...],
                       be_ref[...])


def _finish(mh, mu, var, g, be):
    c = mh.shape[1]
    return pl.pallas_call(
        _finish_body,
        out_shape=jax.ShapeDtypeStruct((N, c), F32),
    )(mh, mu, var, g.reshape(1, c), be.reshape(1, c))


def _edge_conv(featp, d, p0, p1):
    (W1, b1, g1, be1), (W2, b2, g2, be2) = p0, p1
    gw = featp.shape[1]
    c = W1.shape[1]
    W1p = jnp.zeros((2 * gw, c), F32)
    W1p = W1p.at[:d].set(W1[:d]).at[gw:gw + d].set(W1[d:])
    idx = _knn(featp)
    vg = _sc_gather(featp, idx.reshape(-1))
    h1 = _passA(featp, vg, W1p, b1)
    mu1 = jnp.mean(h1, axis=0, keepdims=True)
    var1 = jnp.var(h1, axis=0, keepdims=True)
    h2, mh = _passB(h1, mu1, var1, g1, be1, W2, b2)
    mu2 = jnp.mean(h2, axis=0, keepdims=True)
    var2 = jnp.var(h2, axis=0, keepdims=True)
    return _finish(mh, mu2, var2, g2, be2)


_HEAD_RB = 512


def _head1_body(x1_ref, x2_ref, x3_ref, w_ref, b_ref, y_ref, s_ref, q_ref):
    i = pl.program_id(0)
    w = w_ref[...]
    y = (jnp.dot(x1_ref[...], w[0:64], preferred_element_type=F32)
         + jnp.dot(x2_ref[...], w[64:128], preferred_element_type=F32)
         + jnp.dot(x3_ref[...], w[128:192], preferred_element_type=F32)
         + b_ref[...])
    y = jnp.maximum(y, 0.0)
    y_ref[...] = y

    @pl.when(i == 0)
    def _():
        s_ref[...] = jnp.zeros_like(s_ref)
        q_ref[...] = jnp.zeros_like(q_ref)

    s_ref[...] += jnp.sum(y, axis=0, keepdims=True)
    q_ref[...] += jnp.sum(y * y, axis=0, keepdims=True)


def _head1(x1, x2, x3, W, b):
    co = W.shape[1]
    one = lambda i: (0, 0)
    blk = lambda i: (i, 0)
    return pl.pallas_call(
        _head1_body,
        grid=(N // _HEAD_RB,),
        in_specs=[pl.BlockSpec((_HEAD_RB, 64), blk)] * 3
        + [pl.BlockSpec((192, co), one), pl.BlockSpec((1, co), one)],
        out_specs=(pl.BlockSpec((_HEAD_RB, co), blk),
                   pl.BlockSpec((1, co), one), pl.BlockSpec((1, co), one)),
        out_shape=(jax.ShapeDtypeStruct((N, co), F32),
                   jax.ShapeDtypeStruct((1, co), F32),
                   jax.ShapeDtypeStruct((1, co), F32)),
    )(x1, x2, x3, W, b.reshape(1, co))


def _head_mid_body(y_ref, s_ref, q_ref, g_ref, be_ref, w_ref, b_ref,
                   o_ref, so_ref, qo_ref):
    i = pl.program_id(0)
    mu = s_ref[...] / N
    var = q_ref[...] / N - mu * mu
    z = _bn(y_ref[...], mu, var, g_ref[...], be_ref[...])
    o = jnp.maximum(jnp.dot(z, w_ref[...],
                            preferred_element_type=F32) + b_ref[...], 0.0)
    o_ref[...] = o

    @pl.when(i == 0)
    def _():
        so_ref[...] = jnp.zeros_like(so_ref)
        qo_ref[...] = jnp.zeros_like(qo_ref)

    so_ref[...] += jnp.sum(o, axis=0, keepdims=True)
    qo_ref[...] += jnp.sum(o * o, axis=0, keepdims=True)


def _head_mid(y, s, q, g, be, W, b):
    ci, co = W.shape
    one = lambda i: (0, 0)
    blk = lambda i: (i, 0)
    return pl.pallas_call(
        _head_mid_body,
        grid=(N // _HEAD_RB,),
        in_specs=[
            pl.BlockSpec((_HEAD_RB, ci), blk),
            pl.BlockSpec((1, ci), one), pl.BlockSpec((1, ci), one),
            pl.BlockSpec((1, ci), one), pl.BlockSpec((1, ci), one),
            pl.BlockSpec((ci, co), one), pl.BlockSpec((1, co), one),
        ],
        out_specs=(pl.BlockSpec((_HEAD_RB, co), blk),
                   pl.BlockSpec((1, co), one), pl.BlockSpec((1, co), one)),
        out_shape=(jax.ShapeDtypeStruct((N, co), F32),
                   jax.ShapeDtypeStruct((1, co), F32),
                   jax.ShapeDtypeStruct((1, co), F32)),
    )(y, s, q, g.reshape(1, ci), be.reshape(1, ci), W, b.reshape(1, co))


def _head_final_body(y_ref, s_ref, q_ref, g_ref, be_ref, w_ref, b_ref,
                     o_ref):
    mu = s_ref[...] / N
    var = q_ref[...] / N - mu * mu
    z = _bn(y_ref[...], mu, var, g_ref[...], be_ref[...])
    o = jnp.dot(z, w_ref[...], preferred_element_type=F32) + b_ref[...]
    m = jnp.max(o, axis=1, keepdims=True)
    sh = o - m
    o_ref[...] = sh - jnp.log(jnp.sum(jnp.exp(sh), axis=1, keepdims=True))


def _head_final(y, s, q, g, be, W, b):
    ci, co = W.shape
    one = lambda i: (0, 0)
    blk = lambda i: (i, 0)
    return pl.pallas_call(
        _head_final_body,
        grid=(N // _HEAD_RB,),
        in_specs=[
            pl.BlockSpec((_HEAD_RB, ci), blk),
            pl.BlockSpec((1, ci), one), pl.BlockSpec((1, ci), one),
            pl.BlockSpec((1, ci), one), pl.BlockSpec((1, ci), one),
            pl.BlockSpec((ci, co), one), pl.BlockSpec((1, co), one),
        ],
        out_specs=pl.BlockSpec((_HEAD_RB, co), blk),
        out_shape=jax.ShapeDtypeStruct((N, co), F32),
    )(y, s, q, g.reshape(1, ci), be.reshape(1, ci), W, b.reshape(1, co))


def kernel(x, params):
    p = params
    xp = jnp.pad(x, ((0, 0), (0, 16 - x.shape[1])))
    x1 = _edge_conv(xp, 3, p['c1'][0], p['c1'][1])
    x2 = _edge_conv(x1, 64, p['c2'][0], p['c2'][1])
    x3 = _edge_conv(x2, 64, p['c3'][0], p['c3'][1])
    Wl, bl, gl, bel = p['l1']
    y1, s1, q1 = _head1(x1, x2, x3, Wl, bl)
    Wm1, bm1, gm1, bem1 = p['m1']
    y2, s2, q2 = _head_mid(y1, s1, q1, gl, bel, Wm1, bm1)
    Wm2, bm2, gm2, bem2 = p['m2']
    y3, s3, q3 = _head_mid(y2, s2, q2, gm1, bem1, Wm2, bm2)
    return _head_final(y3, s3, q3, gm2, bem2, p['Wf'], p['bf'])
